# baseline (speedup 1.0000x reference)
import jax
import jax.numpy as jnp
from jax import lax
from jax.experimental import pallas as pl
from jax.experimental.pallas import tpu as pltpu
from jax.experimental.pallas import tpu_sc as plsc

NC = 2
NS = 16
CH = 128
NBUF = 2
NSTG = 2


def _make_agg(n_nodes, d, npad, mh):
    rpt = npad // NS
    last = n_nodes - (NS - 1) * rpt
    assert 0 < last <= rpt and last % 8 == 0
    stage = mh // NSTG
    assert stage % NBUF == 0
    mesh = plsc.VectorSubcoreMesh(core_axis_name="c", subcore_axis_name="s")
    zchunks = [(k * CH, CH) for k in range(rpt // CH)]
    if rpt % CH:
        zchunks.append(((rpt // CH) * CH, rpt % CH))

    def body(h_hbm, src_hbm, dst_hbm, zeros_hbm, out_hbm,
             src_v, dst_v, rows0, rows1, acc, sem0, sem1):
        rows = (rows0, rows1)
        sems = (sem0, sem1)
        c = lax.axis_index("c")
        s = lax.axis_index("s")

        for off, sz in zchunks:
            pltpu.sync_copy(zeros_hbm.at[pl.ds(0, sz)],
                            acc.at[pl.ds(s * rpt + off, sz)])
        plsc.subcore_barrier()

        def start_gather(j, buf, sem):
            pltpu.async_copy(h_hbm.at[src_v.at[j]], buf, sem)

        for st in range(NSTG):
            pltpu.sync_copy(
                src_hbm.at[c].at[s].at[pl.ds(st * stage, stage)], src_v)
            pltpu.sync_copy(
                dst_hbm.at[c].at[s].at[pl.ds(st * stage, stage)], dst_v)
            for k in range(NBUF):
                start_gather(k, rows[k], sems[k])

            def step(i, carry):
                base = NBUF * i
                for k in range(NBUF):
                    j = base + k
                    pltpu.make_async_copy(h_hbm.at[src_v.at[j]],
                                          rows[k], sems[k]).wait()
                    pltpu.sync_copy(rows[k], acc.at[dst_v.at[j]], add=True)

                    @pl.when(j + NBUF < stage)
                    def _():
                        start_gather(j + NBUF, rows[k], sems[k])

                return carry

            lax.fori_loop(0, stage // NBUF, step, 0)

        plsc.subcore_barrier()

        @pl.when(s < NS - 1)
        def _():
            pltpu.sync_copy(acc.at[pl.ds(s * rpt, rpt)],
                            out_hbm.at[c].at[pl.ds(s * rpt, rpt)])

        @pl.when(s == NS - 1)
        def _():
            pltpu.sync_copy(acc.at[pl.ds((NS - 1) * rpt, last)],
                            out_hbm.at[c].at[pl.ds((NS - 1) * rpt, last)])

    return pl.kernel(
        body,
        out_type=jax.ShapeDtypeStruct((NC, n_nodes, d), jnp.float32),
        mesh=mesh,
        scratch_types=[
            pltpu.VMEM((stage, CH), jnp.int32),
            pltpu.VMEM((stage, CH), jnp.int32),
            pltpu.VMEM((CH, d), jnp.float32),
            pltpu.VMEM((CH, d), jnp.float32),
            pltpu.VMEM_SHARED((npad, d), jnp.float32),
            pltpu.SemaphoreType.DMA,
            pltpu.SemaphoreType.DMA,
        ])


def _make_deg(n_nodes, npad, mh):
    rpt = npad // NS
    last = n_nodes - (NS - 1) * rpt
    mesh = plsc.VectorSubcoreMesh(core_axis_name="c", subcore_axis_name="s")
    zchunks = [(k * CH, CH) for k in range(rpt // CH)]
    if rpt % CH:
        zchunks.append(((rpt // CH) * CH, rpt % CH))

    def body(dst_hbm, ones_hbm, zeros_hbm, out_hbm, dst_v, ones_v, dacc):
        c = lax.axis_index("c")
        s = lax.axis_index("s")
        pltpu.sync_copy(dst_hbm.at[c].at[s], dst_v)
        pltpu.sync_copy(ones_hbm, ones_v)
        for off, sz in zchunks:
            pltpu.sync_copy(zeros_hbm.at[pl.ds(0, sz)],
                            dacc.at[pl.ds(s * rpt + off, sz)])
        plsc.subcore_barrier()

        def step(i, carry):
            pltpu.sync_copy(ones_v, dacc.at[dst_v.at[i]], add=True)
            return carry

        lax.fori_loop(0, mh, step, 0)
        plsc.subcore_barrier()

        @pl.when(s < NS - 1)
        def _():
            pltpu.sync_copy(dacc.at[pl.ds(s * rpt, rpt)],
                            out_hbm.at[c].at[pl.ds(s * rpt, rpt)])

        @pl.when(s == NS - 1)
        def _():
            pltpu.sync_copy(dacc.at[pl.ds((NS - 1) * rpt, last)],
                            out_hbm.at[c].at[pl.ds((NS - 1) * rpt, last)])

    return pl.kernel(
        body,
        out_type=jax.ShapeDtypeStruct((NC, n_nodes, 16), jnp.float32),
        compiler_params=pltpu.CompilerParams(use_tc_tiling_on_sc=False),
        mesh=mesh,
        scratch_types=[
            pltpu.VMEM((mh, CH), jnp.int32),
            pltpu.VMEM((CH, 16), jnp.float32),
            pltpu.VMEM_SHARED((npad, 16), jnp.float32),
        ])


def _dgt(a, b):
    return lax.dot_general(a, b, (((1,), (1,)), ((), ())),
                           preferred_element_type=jnp.float32)


def _tc1_body(p0, p1, d0, d1, x, wl, bl, wr, o):
    deg = jnp.maximum(d0[0, :, :1] + d1[0, :, :1], 1.0)
    mean = (p0[0] + p1[0]) / deg
    o[...] = jnp.maximum(_dgt(mean, wl[...]) + bl[...] + _dgt(x[...], wr[...]),
                         0.0)


def _tc2_body(p0, p1, d0, d1, h1, wl, bl, wr, wv, bv, wt, bt, oh, ov, ot):
    deg = jnp.maximum(d0[0, :, :1] + d1[0, :, :1], 1.0)
    mean = (p0[0] + p1[0]) / deg
    h = _dgt(mean, wl[...]) + bl[...] + _dgt(h1[...], wr[...])
    oh[...] = h
    ov[...] = jnp.maximum(_dgt(h, wv[...]) + bv[...], 0.0)
    ot[...] = jnp.maximum(_dgt(h, wt[...]) + bt[...], 0.0)


def _row_spec(bn, w):
    return pl.BlockSpec((bn, w), lambda i: (i, 0))


def _plane_spec(p, bn, w):
    return pl.BlockSpec((1, bn, w), lambda i, _p=p: (_p, i, 0))


def _full_spec():
    return pl.BlockSpec((128, 128), lambda i: (0, 0))


def _bias_spec():
    return pl.BlockSpec((1, 128), lambda i: (0, 0))


def kernel(x, edge_index, Wl1, bl1, Wr1, Wl2, bl2, Wr2, Wv, bv, Wt, bt):
    n, d = x.shape
    e = edge_index.shape[1]
    mh = -(-e // (NC * NS * CH * NBUF * NSTG)) * NBUF * NSTG
    ep = NC * NS * mh * CH
    npad = -(-(n + 1) // (NS * 8)) * (NS * 8)

    pad = ep - e
    srcp = jnp.concatenate(
        [edge_index[0], jnp.zeros((pad,), jnp.int32)]).reshape(NC, NS, mh, CH)
    dstp = jnp.concatenate(
        [edge_index[1], jnp.full((pad,), n, jnp.int32)]).reshape(NC, NS, mh,
                                                                 CH)
    zeros = jnp.zeros((CH, d), jnp.float32)
    ones16 = jnp.ones((CH, 16), jnp.float32)
    zeros16 = jnp.zeros((CH, 16), jnp.float32)

    agg = _make_agg(n, d, npad, mh)
    deg = _make_deg(n, npad, mh)

    degp = deg(dstp, ones16, zeros16)
    agg1 = agg(x, srcp, dstp, zeros)

    bn = 1000
    grid = (n // bn,)
    h1 = pl.pallas_call(
        _tc1_body,
        grid=grid,
        in_specs=[_plane_spec(0, bn, d), _plane_spec(1, bn, d),
                  _plane_spec(0, bn, 16), _plane_spec(1, bn, 16),
                  _row_spec(bn, d), _full_spec(), _bias_spec(), _full_spec()],
        out_specs=_row_spec(bn, d),
        out_shape=jax.ShapeDtypeStruct((n, d), jnp.float32),
    )(agg1, agg1, degp, degp, x, Wl1, bl1.reshape(1, d), Wr1)

    agg2 = agg(h1, srcp, dstp, zeros)

    h, xv, xt = pl.pallas_call(
        _tc2_body,
        grid=grid,
        in_specs=[_plane_spec(0, bn, d), _plane_spec(1, bn, d),
                  _plane_spec(0, bn, 16), _plane_spec(1, bn, 16),
                  _row_spec(bn, d),
                  _full_spec(), _bias_spec(), _full_spec(),
                  _full_spec(), _bias_spec(),
                  _full_spec(), _bias_spec()],
        out_specs=[_row_spec(bn, d), _row_spec(bn, d), _row_spec(bn, d)],
        out_shape=[jax.ShapeDtypeStruct((n, d), jnp.float32),
                   jax.ShapeDtypeStruct((n, d), jnp.float32),
                   jax.ShapeDtypeStruct((n, d), jnp.float32)],
    )(agg2, agg2, degp, degp, h1, Wl2, bl2.reshape(1, d), Wr2,
      Wv, bv.reshape(1, d), Wt, bt.reshape(1, d))

    return (h, xv, xt)


# speedup vs baseline: 1.7802x; 1.7802x over previous
import jax
import jax.numpy as jnp
from jax import lax
from jax.experimental import pallas as pl
from jax.experimental.pallas import tpu as pltpu
from jax.experimental.pallas import tpu_sc as plsc

NC = 2
NS = 16
CH = 128
NBUF = 2


def _make_agg(n_nodes, dh, npad, m, with_deg):
    rpt = npad // NS
    last = n_nodes - (NS - 1) * rpt
    assert 0 < last <= rpt
    mesh = plsc.VectorSubcoreMesh(core_axis_name="c", subcore_axis_name="s")
    zchunks = [(k * CH, CH) for k in range(rpt // CH)]
    if rpt % CH:
        zchunks.append(((rpt // CH) * CH, rpt % CH))

    def body(*refs):
        if with_deg:
            (hs_hbm, src_hbm, dst_hbm, zeros_hbm, ones16_hbm, zeros16_hbm,
             out_hbm, deg_hbm, src_v, dst_v, *rest) = refs
            rows = rest[:NBUF]
            acc = rest[NBUF]
            sems = rest[NBUF + 1:2 * NBUF + 1]
            ones_v, dacc, semd = rest[2 * NBUF + 1:]
        else:
            (hs_hbm, src_hbm, dst_hbm, zeros_hbm,
             out_hbm, src_v, dst_v, *rest) = refs
            rows = rest[:NBUF]
            acc = rest[NBUF]
            sems = rest[NBUF + 1:2 * NBUF + 1]
        c = lax.axis_index("c")
        s = lax.axis_index("s")
        hc_hbm = hs_hbm.at[c]

        pltpu.sync_copy(src_hbm.at[s], src_v)
        pltpu.sync_copy(dst_hbm.at[s], dst_v)
        for off, sz in zchunks:
            pltpu.sync_copy(zeros_hbm.at[pl.ds(0, sz)],
                            acc.at[pl.ds(s * rpt + off, sz)])
        if with_deg:
            pltpu.sync_copy(ones16_hbm, ones_v)
            for off, sz in zchunks:
                pltpu.sync_copy(zeros16_hbm.at[pl.ds(0, sz)],
                                dacc.at[pl.ds(s * rpt + off, sz)])
        plsc.subcore_barrier()

        def start_gather(j, buf, sem):
            pltpu.async_copy(hc_hbm.at[src_v.at[j]], buf, sem)

        for k in range(NBUF):
            start_gather(k, rows[k], sems[k])

        def step(i, carry):
            base = NBUF * i
            for k in range(NBUF):
                j = base + k
                pltpu.make_async_copy(hc_hbm.at[src_v.at[j]],
                                      rows[k], sems[k]).wait()
                pltpu.sync_copy(rows[k], acc.at[dst_v.at[j]], add=True)
                if with_deg:
                    @pl.when(c == (k & 1))
                    def _():
                        pltpu.async_copy(ones_v, dacc.at[dst_v.at[j]],
                                         semd, add=True)

                @pl.when(j + NBUF < m)
                def _():
                    start_gather(j + NBUF, rows[k], sems[k])

            return carry

        lax.fori_loop(0, m // NBUF, step, 0)
        if with_deg:
            def drain(i, carry):
                pltpu.make_async_copy(ones_v, dacc.at[dst_v.at[0]],
                                      semd).wait()
                return carry
            lax.fori_loop(0, m // 2, drain, 0)
        plsc.subcore_barrier()

        @pl.when(s < NS - 1)
        def _():
            pltpu.sync_copy(acc.at[pl.ds(s * rpt, rpt)],
                            out_hbm.at[pl.ds(s * rpt, rpt), pl.ds(c * dh, dh)])

        @pl.when(s == NS - 1)
        def _():
            pltpu.sync_copy(acc.at[pl.ds((NS - 1) * rpt, last)],
                            out_hbm.at[pl.ds((NS - 1) * rpt, last),
                                       pl.ds(c * dh, dh)])

        if with_deg:
            @pl.when(s < NS - 1)
            def _():
                pltpu.sync_copy(dacc.at[pl.ds(s * rpt, rpt)],
                                deg_hbm.at[c].at[pl.ds(s * rpt, rpt)])

            @pl.when(s == NS - 1)
            def _():
                pltpu.sync_copy(dacc.at[pl.ds((NS - 1) * rpt, last)],
                                deg_hbm.at[c].at[pl.ds((NS - 1) * rpt, last)])

    out_type = [jax.ShapeDtypeStruct((n_nodes, 2 * dh), jnp.float32)]
    scratch = [
        pltpu.VMEM((m, CH), jnp.int32),
        pltpu.VMEM((m, CH), jnp.int32),
    ]
    scratch += [pltpu.VMEM((CH, dh), jnp.float32) for _ in range(NBUF)]
    scratch += [pltpu.VMEM_SHARED((npad, dh), jnp.float32)]
    scratch += [pltpu.SemaphoreType.DMA for _ in range(NBUF)]
    if with_deg:
        out_type.append(jax.ShapeDtypeStruct((NC, n_nodes, 16), jnp.float32))
        scratch += [pltpu.VMEM((CH, 16), jnp.float32),
                    pltpu.VMEM_SHARED((npad, 16), jnp.float32),
                    pltpu.SemaphoreType.DMA]

    return pl.kernel(
        body,
        out_type=tuple(out_type) if with_deg else out_type[0],
        compiler_params=pltpu.CompilerParams(use_tc_tiling_on_sc=False),
        mesh=mesh,
        scratch_types=scratch)


def _dgt(a, b):
    return lax.dot_general(a, b, (((1,), (1,)), ((), ())),
                           preferred_element_type=jnp.float32)


def _tc1_body(p, d0, d1, x, wl, bl, wr, o):
    dh = p.shape[1] // 2
    deg = jnp.maximum(d0[0, :, :1] + d1[0, :, :1], 1.0)
    mean = p[...] / deg
    res = jnp.maximum(_dgt(mean, wl[...]) + bl[...] + _dgt(x[...], wr[...]),
                      0.0)
    o[0] = res[:, :dh]
    o[1] = res[:, dh:]


def _tc2_body(p, d0, d1, h1s, wl, bl, wr, wv, bv, wt, bt, oh, ov, ot):
    deg = jnp.maximum(d0[0, :, :1] + d1[0, :, :1], 1.0)
    mean = p[...] / deg
    h1 = jnp.concatenate([h1s[0], h1s[1]], axis=1)
    h = _dgt(mean, wl[...]) + bl[...] + _dgt(h1, wr[...])
    oh[...] = h
    ov[...] = jnp.maximum(_dgt(h, wv[...]) + bv[...], 0.0)
    ot[...] = jnp.maximum(_dgt(h, wt[...]) + bt[...], 0.0)


def _row_spec(bn, w):
    return pl.BlockSpec((bn, w), lambda i: (i, 0))


def _plane_spec(p, bn, w):
    return pl.BlockSpec((1, bn, w), lambda i, _p=p: (_p, i, 0))


def _stk_spec(bn, w):
    return pl.BlockSpec((2, bn, w), lambda i: (0, i, 0))


def _full_spec():
    return pl.BlockSpec((128, 128), lambda i: (0, 0))


def _bias_spec():
    return pl.BlockSpec((1, 128), lambda i: (0, 0))


def kernel(x, edge_index, Wl1, bl1, Wr1, Wl2, bl2, Wr2, Wv, bv, Wt, bt):
    n, d = x.shape
    dh = d // 2
    e = edge_index.shape[1]
    m = -(-e // (NS * CH * NBUF)) * NBUF
    ep = NS * m * CH
    npad = -(-(n + 1) // (NS * 8)) * (NS * 8)

    pad = ep - e
    srcp = jnp.concatenate(
        [edge_index[0], jnp.zeros((pad,), jnp.int32)]).reshape(NS, m, CH)
    dstp = jnp.concatenate(
        [edge_index[1], jnp.full((pad,), n, jnp.int32)]).reshape(NS, m, CH)
    zeros = jnp.zeros((CH, dh), jnp.float32)
    ones16 = jnp.ones((CH, 16), jnp.float32)
    zeros16 = jnp.zeros((CH, 16), jnp.float32)

    agg_deg = _make_agg(n, dh, npad, m, with_deg=True)
    agg = _make_agg(n, dh, npad, m, with_deg=False)

    xs = jnp.stack([x[:, :dh], x[:, dh:]])
    agg1, degp = agg_deg(xs, srcp, dstp, zeros, ones16, zeros16)

    bn = 1000
    grid = (n // bn,)
    h1s = pl.pallas_call(
        _tc1_body,
        grid=grid,
        in_specs=[_row_spec(bn, d),
                  _plane_spec(0, bn, 16), _plane_spec(1, bn, 16),
                  _row_spec(bn, d), _full_spec(), _bias_spec(), _full_spec()],
        out_specs=_stk_spec(bn, dh),
        out_shape=jax.ShapeDtypeStruct((2, n, dh), jnp.float32),
    )(agg1, degp, degp, x, Wl1, bl1.reshape(1, d), Wr1)

    agg2 = agg(h1s, srcp, dstp, zeros)

    h, xv, xt = pl.pallas_call(
        _tc2_body,
        grid=grid,
        in_specs=[_row_spec(bn, d),
                  _plane_spec(0, bn, 16), _plane_spec(1, bn, 16),
                  _stk_spec(bn, dh),
                  _full_spec(), _bias_spec(), _full_spec(),
                  _full_spec(), _bias_spec(),
                  _full_spec(), _bias_spec()],
        out_specs=[_row_spec(bn, d), _row_spec(bn, d), _row_spec(bn, d)],
        out_shape=[jax.ShapeDtypeStruct((n, d), jnp.float32),
                   jax.ShapeDtypeStruct((n, d), jnp.float32),
                   jax.ShapeDtypeStruct((n, d), jnp.float32)],
    )(agg2, degp, degp, h1s, Wl2, bl2.reshape(1, d), Wr2,
      Wv, bv.reshape(1, d), Wt, bt.reshape(1, d))

    return (h, xv, xt)
